# Initial kernel scaffold; baseline (speedup 1.0000x reference)
import jax
import jax.numpy as jnp
from jax.experimental import pallas as pl


def kernel(x, conv_w, conv_b, lin_w, lin_b):
    raise NotImplementedError("write your pallas kernel here")


# speedup vs baseline: 1.1273x; 1.1273x over previous
import jax
import jax.numpy as jnp
from jax import lax
from jax.experimental import pallas as pl
from jax.experimental.pallas import tpu as pltpu


_BT = 16
_NCLS = 10
_PCLS = 128
_CCH = 28
_PSP = 144


def _conv_pool_kernel(p_ref, wc_ref, bc_ref, o_ref):
    w = wc_ref[...]
    c0 = jnp.dot(w, p_ref[0], preferred_element_type=jnp.float32)
    c1 = jnp.dot(w, p_ref[1], preferred_element_type=jnp.float32)
    c2 = jnp.dot(w, p_ref[2], preferred_element_type=jnp.float32)
    c3 = jnp.dot(w, p_ref[3], preferred_element_type=jnp.float32)
    pooled = jnp.maximum(jnp.maximum(c0, c1), jnp.maximum(c2, c3))
    o_ref[...] = jnp.maximum(pooled + bc_ref[...], 0.0).astype(jnp.bfloat16)


def _linear_lsm_kernel(f_ref, wl_ref, bl_ref, o_ref):
    part = lax.dot_general(
        f_ref[...], wl_ref[...],
        dimension_numbers=(((2,), (1,)), ((0,), (0,))),
        preferred_element_type=jnp.float32)
    logits = jnp.sum(part, axis=0) + bl_ref[...]
    m = jnp.max(logits, axis=-1, keepdims=True)
    s = logits - m
    lse = jnp.log(jnp.sum(jnp.exp(s), axis=-1, keepdims=True))
    o_ref[...] = s - lse


def _im2col(x):
    n = x.shape[0]
    xs = x[:, 0]
    offs = []
    for a in range(2):
        for b in range(2):
            taps = []
            for kh in range(5):
                for kw in range(5):
                    win = xs[:, a + kh: a + kh + 24: 2, b + kw: b + kw + 24: 2]
                    taps.append(win.reshape(n, _PSP))
            offs.append(jnp.stack(taps, axis=0))
    return jnp.stack(offs, axis=0).reshape(4, 25, n * _PSP).astype(jnp.bfloat16)


@jax.jit
def _forward(x, conv_w, conv_b, lin_w, lin_b):
    n = x.shape[0]
    bt = _BT
    n_pad = ((n + bt - 1) // bt) * bt
    x = x.astype(jnp.float32)
    if n_pad != n:
        x = jnp.pad(x, ((0, n_pad - n), (0, 0), (0, 0), (0, 0)))

    patches = _im2col(x)
    wc = conv_w.reshape(_CCH, 25).astype(jnp.bfloat16)
    bc = conv_b.reshape(_CCH, 1).astype(jnp.float32)

    grid = (n_pad // bt,)
    cparams = pltpu.CompilerParams(
        dimension_semantics=("parallel",),
        vmem_limit_bytes=64 * 1024 * 1024)

    feats = pl.pallas_call(
        _conv_pool_kernel,
        grid=grid,
        in_specs=[
            pl.BlockSpec((4, 25, bt * _PSP), lambda i: (0, 0, i)),
            pl.BlockSpec((_CCH, 25), lambda i: (0, 0)),
            pl.BlockSpec((_CCH, 1), lambda i: (0, 0)),
        ],
        out_specs=pl.BlockSpec((_CCH, bt * _PSP), lambda i: (0, i)),
        out_shape=jax.ShapeDtypeStruct((_CCH, n_pad * _PSP), jnp.bfloat16),
        compiler_params=cparams,
    )(patches, wc, bc)

    feats3 = feats.reshape(_CCH, n_pad, _PSP)

    wl3 = lin_w.astype(jnp.float32).reshape(_NCLS, _CCH, _PSP)
    wl3 = jnp.transpose(wl3, (1, 2, 0))
    wl3 = jnp.pad(wl3, ((0, 0), (0, 0), (0, _PCLS - _NCLS))).astype(jnp.bfloat16)
    bl = jnp.pad(lin_b.astype(jnp.float32), (0, _PCLS - _NCLS),
                 constant_values=-1e30).reshape(1, _PCLS)

    out = pl.pallas_call(
        _linear_lsm_kernel,
        grid=grid,
        in_specs=[
            pl.BlockSpec((_CCH, bt, _PSP), lambda i: (0, i, 0)),
            pl.BlockSpec((_CCH, _PSP, _PCLS), lambda i: (0, 0, 0)),
            pl.BlockSpec((1, _PCLS), lambda i: (0, 0)),
        ],
        out_specs=pl.BlockSpec((bt, _PCLS), lambda i: (i, 0)),
        out_shape=jax.ShapeDtypeStruct((n_pad, _PCLS), jnp.float32),
        compiler_params=cparams,
    )(feats3, wl3, bl)

    return out[:n, :_NCLS]


def kernel(x, conv_w, conv_b, lin_w, lin_b):
    return _forward(x, conv_w, conv_b, lin_w, lin_b)


# speedup vs baseline: 22.7299x; 20.1631x over previous
import functools

import jax
import jax.numpy as jnp
from jax import lax
from jax.experimental import pallas as pl
from jax.experimental.pallas import tpu as pltpu


_BT = 128
_NCLS = 10
_PCLS = 128
_CCH = 28
_PSP = 144


def _fused_kernel(xq_ref, wext_ref, bc_ref, wl_ref, bl_ref, o_ref):
    bt = o_ref.shape[0]
    xq = xq_ref[...]

    wins = []
    for ph in range(2):
        for pw in range(2):
            base = (ph * 2 + pw) * 14
            for bh in range(3):
                for bw in range(3):
                    wins.append(xq[base + bh: base + bh + 12, bw: bw + 12, :])
    pool = jnp.stack(wins, axis=0).reshape(36, _PSP, bt).astype(jnp.bfloat16)

    conv = lax.dot_general(
        wext_ref[...], pool,
        dimension_numbers=(((1,), (0,)), ((), ())),
        preferred_element_type=jnp.float32)

    pooled = jnp.maximum(jnp.maximum(conv[0:28], conv[28:56]),
                         jnp.maximum(conv[56:84], conv[84:112]))
    feats = jnp.maximum(pooled + bc_ref[...].reshape(_CCH, 1, 1), 0.0)
    feats = feats.astype(jnp.bfloat16).reshape(_CCH * _PSP, bt)

    logits = lax.dot_general(
        feats, wl_ref[...],
        dimension_numbers=(((0,), (0,)), ((), ())),
        preferred_element_type=jnp.float32)
    logits = logits + bl_ref[...]
    m = jnp.max(logits, axis=-1, keepdims=True)
    s = logits - m
    lse = jnp.log(jnp.sum(jnp.exp(s), axis=-1, keepdims=True))
    o_ref[...] = s - lse


def _build_wext(conv_w):
    w = conv_w.reshape(_CCH, 5, 5)
    cols = []
    rows = []
    for a in range(2):
        for b in range(2):
            blk = jnp.zeros((_CCH, 36), conv_w.dtype)
            for kh in range(5):
                for kw in range(5):
                    v, wv = a + kh, b + kw
                    u = (v % 2) * 18 + (wv % 2) * 9 + (v // 2) * 3 + (wv // 2)
                    blk = blk.at[:, u].set(w[:, kh, kw])
            rows.append(blk)
    del cols
    return jnp.concatenate(rows, axis=0)


@functools.partial(jax.jit, static_argnames=())
def _forward(x, conv_w, conv_b, lin_w, lin_b):
    n = x.shape[0]
    bt = _BT
    n_pad = ((n + bt - 1) // bt) * bt
    x = x.astype(jnp.float32)
    if n_pad != n:
        x = jnp.pad(x, ((0, n_pad - n), (0, 0), (0, 0), (0, 0)))

    xs = x[:, 0].reshape(n_pad, 14, 2, 14, 2)
    xq = jnp.transpose(xs, (2, 4, 1, 3, 0)).reshape(56, 14, n_pad)

    wext = _build_wext(conv_w).astype(jnp.bfloat16)
    bc = conv_b.reshape(_CCH, 1).astype(jnp.float32)
    wl = jnp.pad(lin_w.astype(jnp.float32).T,
                 ((0, 0), (0, _PCLS - _NCLS))).astype(jnp.bfloat16)
    bl = jnp.pad(lin_b.astype(jnp.float32), (0, _PCLS - _NCLS),
                 constant_values=-1e30).reshape(1, _PCLS)

    grid = (n_pad // bt,)
    out = pl.pallas_call(
        _fused_kernel,
        grid=grid,
        in_specs=[
            pl.BlockSpec((56, 14, bt), lambda i: (0, 0, i)),
            pl.BlockSpec((112, 36), lambda i: (0, 0)),
            pl.BlockSpec((_CCH, 1), lambda i: (0, 0)),
            pl.BlockSpec((_CCH * _PSP, _PCLS), lambda i: (0, 0)),
            pl.BlockSpec((1, _PCLS), lambda i: (0, 0)),
        ],
        out_specs=pl.BlockSpec((bt, _PCLS), lambda i: (i, 0)),
        out_shape=jax.ShapeDtypeStruct((n_pad, _PCLS), jnp.float32),
        compiler_params=pltpu.CompilerParams(
            dimension_semantics=("parallel",),
            vmem_limit_bytes=64 * 1024 * 1024),
    )(xq, wext, bc, wl, bl)

    return out[:n, :_NCLS]


def kernel(x, conv_w, conv_b, lin_w, lin_b):
    return _forward(x, conv_w, conv_b, lin_w, lin_b)


# speedup vs baseline: 23.0113x; 1.0124x over previous
import functools

import jax
import jax.numpy as jnp
from jax import lax
from jax.experimental import pallas as pl
from jax.experimental.pallas import tpu as pltpu


_BT = 256
_NCLS = 10
_PCLS = 128
_CCH = 28
_PSP = 144


def _fused_kernel(xq_ref, wext_ref, bc_ref, wl_ref, bl_ref, o_ref):
    bt = o_ref.shape[0]
    xq = xq_ref[...]

    wins = []
    for ph in range(2):
        for pw in range(2):
            base = (ph * 2 + pw) * 14
            for bh in range(3):
                for bw in range(3):
                    wins.append(xq[base + bh: base + bh + 12, bw: bw + 12, :])
    pool = jnp.stack(wins, axis=0).reshape(36, _PSP, bt).astype(jnp.bfloat16)

    conv = lax.dot_general(
        wext_ref[...], pool,
        dimension_numbers=(((1,), (0,)), ((), ())),
        preferred_element_type=jnp.float32)

    pooled = jnp.maximum(jnp.maximum(conv[0:28], conv[28:56]),
                         jnp.maximum(conv[56:84], conv[84:112]))
    feats = jnp.maximum(pooled + bc_ref[...].reshape(_CCH, 1, 1), 0.0)
    feats = feats.astype(jnp.bfloat16).reshape(_CCH * _PSP, bt)

    logits = lax.dot_general(
        feats, wl_ref[...],
        dimension_numbers=(((0,), (0,)), ((), ())),
        preferred_element_type=jnp.float32)
    logits = logits + bl_ref[...]
    m = jnp.max(logits, axis=-1, keepdims=True)
    s = logits - m
    lse = jnp.log(jnp.sum(jnp.exp(s), axis=-1, keepdims=True))
    o_ref[...] = s - lse


def _build_wext(conv_w):
    w = conv_w.reshape(_CCH, 5, 5)
    cols = []
    rows = []
    for a in range(2):
        for b in range(2):
            blk = jnp.zeros((_CCH, 36), conv_w.dtype)
            for kh in range(5):
                for kw in range(5):
                    v, wv = a + kh, b + kw
                    u = (v % 2) * 18 + (wv % 2) * 9 + (v // 2) * 3 + (wv // 2)
                    blk = blk.at[:, u].set(w[:, kh, kw])
            rows.append(blk)
    del cols
    return jnp.concatenate(rows, axis=0)


@functools.partial(jax.jit, static_argnames=())
def _forward(x, conv_w, conv_b, lin_w, lin_b):
    n = x.shape[0]
    bt = _BT
    n_pad = ((n + bt - 1) // bt) * bt
    x = x.astype(jnp.float32)
    if n_pad != n:
        x = jnp.pad(x, ((0, n_pad - n), (0, 0), (0, 0), (0, 0)))

    xs = x[:, 0].reshape(n_pad, 14, 2, 14, 2)
    xq = jnp.transpose(xs, (2, 4, 1, 3, 0)).reshape(56, 14, n_pad)

    wext = _build_wext(conv_w).astype(jnp.bfloat16)
    bc = conv_b.reshape(_CCH, 1).astype(jnp.float32)
    wl = jnp.pad(lin_w.astype(jnp.float32).T,
                 ((0, 0), (0, _PCLS - _NCLS))).astype(jnp.bfloat16)
    bl = jnp.pad(lin_b.astype(jnp.float32), (0, _PCLS - _NCLS),
                 constant_values=-1e30).reshape(1, _PCLS)

    grid = (n_pad // bt,)
    out = pl.pallas_call(
        _fused_kernel,
        grid=grid,
        in_specs=[
            pl.BlockSpec((56, 14, bt), lambda i: (0, 0, i)),
            pl.BlockSpec((112, 36), lambda i: (0, 0)),
            pl.BlockSpec((_CCH, 1), lambda i: (0, 0)),
            pl.BlockSpec((_CCH * _PSP, _PCLS), lambda i: (0, 0)),
            pl.BlockSpec((1, _PCLS), lambda i: (0, 0)),
        ],
        out_specs=pl.BlockSpec((bt, _PCLS), lambda i: (i, 0)),
        out_shape=jax.ShapeDtypeStruct((n_pad, _PCLS), jnp.float32),
        compiler_params=pltpu.CompilerParams(
            dimension_semantics=("parallel",),
            vmem_limit_bytes=64 * 1024 * 1024),
    )(xq, wext, bc, wl, bl)

    return out[:n, :_NCLS]


def kernel(x, conv_w, conv_b, lin_w, lin_b):
    return _forward(x, conv_w, conv_b, lin_w, lin_b)


# speedup vs baseline: 23.4598x; 1.0195x over previous
import functools

import jax
import jax.numpy as jnp
from jax import lax
from jax.experimental import pallas as pl
from jax.experimental.pallas import tpu as pltpu


_BT = 256
_NCLS = 10
_PCLS = 128
_CCH = 28
_PSP = 144


def _fused_kernel(xq_ref, wext_ref, bc_ref, wl_ref, bl_ref, o_ref):
    bt = o_ref.shape[0]
    xq = xq_ref[...]

    wins = []
    for ph in range(2):
        for pw in range(2):
            base = (ph * 2 + pw) * 14
            for bh in range(3):
                for bw in range(3):
                    wins.append(xq[base + bh: base + bh + 12, bw: bw + 12, :])
    pool = jnp.stack(wins, axis=0).reshape(36, _PSP, bt)

    conv = lax.dot_general(
        wext_ref[...], pool,
        dimension_numbers=(((1,), (0,)), ((), ())),
        preferred_element_type=jnp.float32)

    pooled = jnp.maximum(jnp.maximum(conv[0:28], conv[28:56]),
                         jnp.maximum(conv[56:84], conv[84:112]))
    feats = jnp.maximum(pooled + bc_ref[...].reshape(_CCH, 1, 1), 0.0)
    feats = feats.astype(jnp.bfloat16).reshape(_CCH * _PSP, bt)

    logits = lax.dot_general(
        feats, wl_ref[...],
        dimension_numbers=(((0,), (0,)), ((), ())),
        preferred_element_type=jnp.float32)
    logits = logits + bl_ref[...]
    m = jnp.max(logits, axis=-1, keepdims=True)
    s = logits - m
    lse = jnp.log(jnp.sum(jnp.exp(s), axis=-1, keepdims=True))
    o_ref[...] = s - lse


def _build_wext(conv_w):
    w = conv_w.reshape(_CCH, 5, 5)
    cols = []
    rows = []
    for a in range(2):
        for b in range(2):
            blk = jnp.zeros((_CCH, 36), conv_w.dtype)
            for kh in range(5):
                for kw in range(5):
                    v, wv = a + kh, b + kw
                    u = (v % 2) * 18 + (wv % 2) * 9 + (v // 2) * 3 + (wv // 2)
                    blk = blk.at[:, u].set(w[:, kh, kw])
            rows.append(blk)
    del cols
    return jnp.concatenate(rows, axis=0)


@functools.partial(jax.jit, static_argnames=())
def _forward(x, conv_w, conv_b, lin_w, lin_b):
    n = x.shape[0]
    bt = _BT
    n_pad = ((n + bt - 1) // bt) * bt
    x = x.astype(jnp.float32)
    if n_pad != n:
        x = jnp.pad(x, ((0, n_pad - n), (0, 0), (0, 0), (0, 0)))

    xs = x[:, 0].astype(jnp.bfloat16).reshape(n_pad, 14, 2, 14, 2)
    xq = jnp.transpose(xs, (2, 4, 1, 3, 0)).reshape(56, 14, n_pad)

    wext = _build_wext(conv_w).astype(jnp.bfloat16)
    bc = conv_b.reshape(_CCH, 1).astype(jnp.float32)
    wl = jnp.pad(lin_w.astype(jnp.float32).T,
                 ((0, 0), (0, _PCLS - _NCLS))).astype(jnp.bfloat16)
    bl = jnp.pad(lin_b.astype(jnp.float32), (0, _PCLS - _NCLS),
                 constant_values=-1e30).reshape(1, _PCLS)

    grid = (n_pad // bt,)
    out = pl.pallas_call(
        _fused_kernel,
        grid=grid,
        in_specs=[
            pl.BlockSpec((56, 14, bt), lambda i: (0, 0, i)),
            pl.BlockSpec((112, 36), lambda i: (0, 0)),
            pl.BlockSpec((_CCH, 1), lambda i: (0, 0)),
            pl.BlockSpec((_CCH * _PSP, _PCLS), lambda i: (0, 0)),
            pl.BlockSpec((1, _PCLS), lambda i: (0, 0)),
        ],
        out_specs=pl.BlockSpec((bt, _PCLS), lambda i: (i, 0)),
        out_shape=jax.ShapeDtypeStruct((n_pad, _PCLS), jnp.float32),
        compiler_params=pltpu.CompilerParams(
            dimension_semantics=("parallel",),
            vmem_limit_bytes=64 * 1024 * 1024),
    )(xq, wext, bc, wl, bl)

    return out[:n, :_NCLS]


def kernel(x, conv_w, conv_b, lin_w, lin_b):
    return _forward(x, conv_w, conv_b, lin_w, lin_b)
